# baseline (speedup 1.0000x reference)
import jax
import jax.numpy as jnp
from jax.experimental import pallas as pl


def _add_kernel(x_ref, e_ref, o_ref):
    o_ref[...] = x_ref[...] + e_ref[...]


def kernel(encoded_patches, position_embedding):
    B, P, D = encoded_patches.shape
    PD = P * D
    x2 = encoded_patches.reshape(B, PD)
    e2 = position_embedding.reshape(1, PD)
    BB = 8
    out2 = pl.pallas_call(
        _add_kernel,
        grid=(B // BB,),
        in_specs=[
            pl.BlockSpec((BB, PD), lambda i: (i, 0)),
            pl.BlockSpec((1, PD), lambda i: (0, 0)),
        ],
        out_specs=pl.BlockSpec((BB, PD), lambda i: (i, 0)),
        out_shape=jax.ShapeDtypeStruct((B, PD), jnp.float32),
    )(x2, e2)
    return out2.reshape(B, P, D)


# speedup vs baseline: 1.0060x; 1.0060x over previous
import jax
import jax.numpy as jnp
from jax.experimental import pallas as pl
from jax.experimental.pallas import tpu as pltpu


def _make_stream_kernel(B, PD, CB, K, M):
    NCHUNK = B // CB

    def _stream_kernel(x_hbm, e_vmem, o_hbm, buf_in, buf_out, in_sem, out_sem):
        def in_copy(c):
            return pltpu.make_async_copy(
                x_hbm.at[pl.ds(c * CB, CB), :], buf_in.at[c % K], in_sem.at[c % K]
            )

        def out_copy(c):
            return pltpu.make_async_copy(
                buf_out.at[c % M], o_hbm.at[pl.ds(c * CB, CB), :], out_sem.at[c % M]
            )

        for c in range(min(K, NCHUNK)):
            in_copy(c).start()
        for c in range(NCHUNK):
            in_copy(c).wait()
            if c >= M:
                out_copy(c - M).wait()
            buf_out[c % M] = buf_in[c % K] + e_vmem[...]
            out_copy(c).start()
            if c + K < NCHUNK:
                in_copy(c + K).start()
        for c in range(max(NCHUNK - M, 0), NCHUNK):
            out_copy(c).wait()

    return _stream_kernel


def kernel(encoded_patches, position_embedding):
    B, P, D = encoded_patches.shape
    PD = P * D
    x2 = encoded_patches.reshape(B, PD)
    e2 = position_embedding.reshape(1, PD)
    CB = 8
    K = 4
    M = 4
    out2 = pl.pallas_call(
        _make_stream_kernel(B, PD, CB, K, M),
        in_specs=[
            pl.BlockSpec(memory_space=pltpu.MemorySpace.HBM),
            pl.BlockSpec(memory_space=pltpu.MemorySpace.VMEM),
        ],
        out_specs=pl.BlockSpec(memory_space=pltpu.MemorySpace.HBM),
        out_shape=jax.ShapeDtypeStruct((B, PD), jnp.float32),
        scratch_shapes=[
            pltpu.MemorySpace.VMEM((K, CB, PD), jnp.float32),
            pltpu.MemorySpace.VMEM((M, CB, PD), jnp.float32),
            pltpu.SemaphoreType.DMA((K,)),
            pltpu.SemaphoreType.DMA((M,)),
        ],
    )(x2, e2)
    return out2.reshape(B, P, D)
